# baseline (speedup 1.0000x reference)
import functools

import jax
import jax.numpy as jnp
import numpy as np
from jax import lax
from jax.experimental import pallas as pl
from jax.experimental.pallas import tpu as pltpu
from jax.experimental.pallas import tpu_sc as plsc

D_ = 256
B_ = 16384
L_ = 16

NC_ = 2
NS_ = 16
NW_ = NC_ * NS_
CH_ = 64

_LO_IDX = np.array([32 * (w // L_) + (w % L_) for w in range(D_ // 2)],
                   dtype=np.int32)
_HI_IDX = _LO_IDX + L_


def _sc_gather_sqdiff(p_idx, q_idx, table):
    n = p_idx.shape[0]
    pairs_per_w = n // NW_
    n_chunks = pairs_per_w // CH_
    mesh = plsc.VectorSubcoreMesh(
        core_axis_name="c", subcore_axis_name="s",
        num_cores=NC_, num_subcores=NS_)

    @functools.partial(
        pl.kernel,
        out_type=jax.ShapeDtypeStruct((n, D_ // 2), jnp.int32),
        mesh=mesh,
        scratch_types=[
            pltpu.VMEM((pairs_per_w,), jnp.int32),
            pltpu.VMEM((pairs_per_w,), jnp.int32),
            pltpu.VMEM((CH_, D_), jnp.float32),
            pltpu.VMEM((CH_, D_), jnp.float32),
            pltpu.VMEM((CH_, D_), jnp.float32),
            pltpu.VMEM((CH_, D_), jnp.float32),
            pltpu.VMEM((CH_, D_ // 2), jnp.int32),
            pltpu.VMEM((CH_, D_ // 2), jnp.int32),
            pltpu.SemaphoreType.DMA,
            pltpu.SemaphoreType.DMA,
            pltpu.SemaphoreType.DMA,
            pltpu.SemaphoreType.DMA,
        ],
        compiler_params=pltpu.CompilerParams(needs_layout_passes=False),
    )
    def gk(pidx_hbm, qidx_hbm, tab_hbm, out_hbm,
           pidx_v, qidx_v, bp0, bp1, bq0, bq1, xb0, xb1, gs0, gs1, ss0, ss1):
        wid = lax.axis_index("s") * NC_ + lax.axis_index("c")
        base = wid * pairs_per_w
        bp = (bp0, bp1)
        bq = (bq0, bq1)
        xb = (xb0, xb1)
        gsem = (gs0, gs1)
        ssem = (ss0, ss1)

        pltpu.sync_copy(pidx_hbm.at[pl.ds(base, pairs_per_w)], pidx_v)
        pltpu.sync_copy(qidx_hbm.at[pl.ds(base, pairs_per_w)], qidx_v)

        def fire_gather(c, s):
            isl = pl.ds(c * CH_, CH_)
            hp = pltpu.async_copy(tab_hbm.at[pidx_v.at[isl]], bp[s], gsem[s])
            hq = pltpu.async_copy(tab_hbm.at[qidx_v.at[isl]], bq[s], gsem[s])
            return (hp, hq)

        def compute(s):
            bps, bqs, xbs = bp[s], bq[s], xb[s]

            @plsc.parallel_loop(0, CH_)
            def _(r):
                for g in range(D_ // (2 * L_)):
                    sla = pl.ds(32 * g, L_)
                    slb = pl.ds(32 * g + L_, L_)
                    da = bps[r, sla] - bqs[r, sla]
                    db = bps[r, slb] - bqs[r, slb]
                    ua = plsc.bitcast(da * da, jnp.int32)
                    ub = plsc.bitcast(db * db, jnp.int32)
                    xbs[r, pl.ds(L_ * g, L_)] = (
                        ((ua + 0x8000) >> 16)
                        | ((ub + 0x8000) & jnp.int32(-65536)))

        pend = [None, None]
        drain = [None, None]
        pend[0] = fire_gather(0, 0)
        for c in range(n_chunks):
            s = c % 2
            o = 1 - s
            for h in pend[s]:
                h.wait()
            if c + 1 < n_chunks:
                pend[o] = fire_gather(c + 1, o)
            if drain[s] is not None:
                drain[s].wait()
            compute(s)
            drain[s] = pltpu.async_copy(
                xb[s], out_hbm.at[pl.ds(base + c * CH_, CH_)], ssem[s])
        drain[0].wait()
        drain[1].wait()

    return gk(p_idx, q_idx, table)


BB_ = 4096


def _mlp_body(xw_ref, w1g_ref, b1_ref, w2r_ref, b2_ref, out_ref):
    xi = xw_ref[...]
    xlo = jax.lax.bitcast_convert_type(
        xi << 16, jnp.float32).astype(jnp.bfloat16)
    xhi = jax.lax.bitcast_convert_type(
        xi & jnp.int32(-65536), jnp.float32).astype(jnp.bfloat16)
    h = (jnp.dot(xlo, w1g_ref[0], preferred_element_type=jnp.float32)
         + jnp.dot(xhi, w1g_ref[1], preferred_element_type=jnp.float32))
    h = jnp.maximum(h + b1_ref[...], 0.0)
    psum = jnp.sum(h * w2r_ref[...], axis=1, keepdims=True) + b2_ref[0, 0]
    out_ref[...] = jnp.transpose(psum, (1, 0))


def _tc_mlp(xw, W1g, b1, W2r, b2):
    n = xw.shape[0]
    nb = n // BB_
    return pl.pallas_call(
        _mlp_body,
        grid=(nb,),
        in_specs=[
            pl.BlockSpec((BB_, D_ // 2), lambda i: (i, 0)),
            pl.BlockSpec((2, D_ // 2, D_), lambda i: (0, 0, 0)),
            pl.BlockSpec((1, D_), lambda i: (0, 0)),
            pl.BlockSpec((1, D_), lambda i: (0, 0)),
            pl.BlockSpec(memory_space=pltpu.SMEM),
        ],
        out_specs=pl.BlockSpec((1, BB_), lambda i: (0, i)),
        out_shape=jax.ShapeDtypeStruct((1, n), jnp.float32),
    )(xw, W1g, b1.reshape(1, D_), W2r, b2.reshape(1, 1))


def kernel(p_vertices, q_vertices, embds, W1, b1, W2, b2):
    W1g = (W1.reshape(D_ // (2 * L_), 2, L_, D_)
           .transpose(1, 0, 2, 3).reshape(2, D_ // 2, D_)
           .astype(jnp.bfloat16))
    W2r = W2.reshape(1, D_)
    h = B_ // 2
    preds = []
    for lo in (0, h):
        xw = _sc_gather_sqdiff(
            jax.lax.slice(p_vertices, (lo,), (lo + h,)),
            jax.lax.slice(q_vertices, (lo,), (lo + h,)), embds)
        preds.append(_tc_mlp(xw, W1g, b1, W2r, b2)[0])
    return jnp.concatenate(preds)


# speedup vs baseline: 1.0669x; 1.0669x over previous
import functools

import jax
import jax.numpy as jnp
import numpy as np
from jax import lax
from jax.experimental import pallas as pl
from jax.experimental.pallas import tpu as pltpu
from jax.experimental.pallas import tpu_sc as plsc

D_ = 256
B_ = 16384
L_ = 16

NC_ = 2
NS_ = 16
NW_ = NC_ * NS_
PAIRS_PER_W_ = B_ // NW_
CH_ = 64
N_CHUNKS_ = PAIRS_PER_W_ // CH_

_LO_IDX = np.array([32 * (w // L_) + (w % L_) for w in range(D_ // 2)],
                   dtype=np.int32)
_HI_IDX = _LO_IDX + L_


def _sc_gather_sqdiff(p_idx, q_idx, table):
    mesh = plsc.VectorSubcoreMesh(
        core_axis_name="c", subcore_axis_name="s",
        num_cores=NC_, num_subcores=NS_)

    @functools.partial(
        pl.kernel,
        out_type=jax.ShapeDtypeStruct((B_, D_ // 2), jnp.int32),
        mesh=mesh,
        scratch_types=[
            pltpu.VMEM((PAIRS_PER_W_,), jnp.int32),
            pltpu.VMEM((PAIRS_PER_W_,), jnp.int32),
            pltpu.VMEM((CH_, D_), jnp.float32),
            pltpu.VMEM((CH_, D_), jnp.float32),
            pltpu.VMEM((CH_, D_), jnp.float32),
            pltpu.VMEM((CH_, D_), jnp.float32),
            pltpu.VMEM((CH_, D_ // 2), jnp.int32),
            pltpu.VMEM((CH_, D_ // 2), jnp.int32),
            pltpu.SemaphoreType.DMA,
            pltpu.SemaphoreType.DMA,
            pltpu.SemaphoreType.DMA,
            pltpu.SemaphoreType.DMA,
        ],
        compiler_params=pltpu.CompilerParams(needs_layout_passes=False),
    )
    def gk(pidx_hbm, qidx_hbm, tab_hbm, out_hbm,
           pidx_v, qidx_v, bp0, bp1, bq0, bq1, xb0, xb1, gs0, gs1, ss0, ss1):
        wid = lax.axis_index("s") * NC_ + lax.axis_index("c")
        base = wid * PAIRS_PER_W_
        bp = (bp0, bp1)
        bq = (bq0, bq1)
        xb = (xb0, xb1)
        gsem = (gs0, gs1)
        ssem = (ss0, ss1)

        pltpu.sync_copy(pidx_hbm.at[pl.ds(base, PAIRS_PER_W_)], pidx_v)
        pltpu.sync_copy(qidx_hbm.at[pl.ds(base, PAIRS_PER_W_)], qidx_v)

        def fire_gather(c, s):
            isl = pl.ds(c * CH_, CH_)
            hp = pltpu.async_copy(tab_hbm.at[pidx_v.at[isl]], bp[s], gsem[s])
            hq = pltpu.async_copy(tab_hbm.at[qidx_v.at[isl]], bq[s], gsem[s])
            return (hp, hq)

        def compute(s):
            bps, bqs, xbs = bp[s], bq[s], xb[s]

            @plsc.parallel_loop(0, CH_)
            def _(r):
                for g in range(D_ // (2 * L_)):
                    sla = pl.ds(32 * g, L_)
                    slb = pl.ds(32 * g + L_, L_)
                    da = bps[r, sla] - bqs[r, sla]
                    db = bps[r, slb] - bqs[r, slb]
                    ua = plsc.bitcast(da * da, jnp.int32)
                    ub = plsc.bitcast(db * db, jnp.int32)
                    xbs[r, pl.ds(L_ * g, L_)] = (
                        ((ua + 0x8000) >> 16)
                        | ((ub + 0x8000) & jnp.int32(-65536)))

        pend = [None, None]
        drain = [None, None]
        pend[0] = fire_gather(0, 0)
        for c in range(N_CHUNKS_):
            s = c % 2
            o = 1 - s
            for h in pend[s]:
                h.wait()
            if c + 1 < N_CHUNKS_:
                pend[o] = fire_gather(c + 1, o)
            if drain[s] is not None:
                drain[s].wait()
            compute(s)
            drain[s] = pltpu.async_copy(
                xb[s], out_hbm.at[pl.ds(base + c * CH_, CH_)], ssem[s])
        drain[0].wait()
        drain[1].wait()

    return gk(p_idx, q_idx, table)


BB_ = 4096


def _mlp_body(xw_ref, w1g_ref, b1_ref, w2r_ref, b2_ref, out_ref):
    xi = xw_ref[...]
    xlo = jax.lax.bitcast_convert_type(
        xi << 16, jnp.float32).astype(jnp.bfloat16)
    xhi = jax.lax.bitcast_convert_type(
        xi & jnp.int32(-65536), jnp.float32).astype(jnp.bfloat16)
    h = (jnp.dot(xlo, w1g_ref[0], preferred_element_type=jnp.float32)
         + jnp.dot(xhi, w1g_ref[1], preferred_element_type=jnp.float32))
    h = jnp.maximum(h + b1_ref[...], 0.0)
    psum = jnp.sum(h * w2r_ref[...], axis=1, keepdims=True) + b2_ref[0, 0]
    out_ref[...] = jnp.transpose(psum, (1, 0))


def _tc_mlp(xw, W1g, b1, W2r, b2):
    nb = B_ // BB_
    return pl.pallas_call(
        _mlp_body,
        grid=(nb,),
        in_specs=[
            pl.BlockSpec((BB_, D_ // 2), lambda i: (i, 0)),
            pl.BlockSpec((2, D_ // 2, D_), lambda i: (0, 0, 0)),
            pl.BlockSpec((1, D_), lambda i: (0, 0)),
            pl.BlockSpec((1, D_), lambda i: (0, 0)),
            pl.BlockSpec(memory_space=pltpu.SMEM),
        ],
        out_specs=pl.BlockSpec((1, BB_), lambda i: (0, i)),
        out_shape=jax.ShapeDtypeStruct((1, B_), jnp.float32),
    )(xw, W1g, b1.reshape(1, D_), W2r, b2.reshape(1, 1))


def kernel(p_vertices, q_vertices, embds, W1, b1, W2, b2):
    xw = _sc_gather_sqdiff(p_vertices, q_vertices, embds)
    W1g = (W1.reshape(D_ // (2 * L_), 2, L_, D_)
           .transpose(1, 0, 2, 3).reshape(2, D_ // 2, D_)
           .astype(jnp.bfloat16))
    return _tc_mlp(xw, W1g, b1, W2.reshape(1, D_), b2)[0]
